# baseline (speedup 1.0000x reference)
import dataclasses
import functools

import jax
import jax.numpy as jnp
from jax import lax
from jax.experimental import pallas as pl
from jax.experimental.pallas import tpu as pltpu
from jax.experimental.pallas import tpu_sc as plsc

N = 10000
D = 128
H = 128
E = 320000

NC = 2
NS = 16
NW = NC * NS

FW = 64
NQ = 2
NP = NQ // NC
CW = 128
CPT = 160
EPAD = NS * CPT * CW
KB = 16
NB = CPT // KB
NBUF = 3
LAG = 2
NPAD = 10240
RPT = NPAD // (NS * CW)

BLK = 512
ER = E // CW
ERP = NS * CPT
EB = ERP // (NPAD // BLK)
FBLK = 400


def _sc_body(tabs, srcr, dstr, z32, oagg, sidx, didx, rows, zbuf, gsems,
             ssems, tab_sp, acc, ocnt=None, cnt_local=None):
    cid = lax.axis_index("c")
    sid = lax.axis_index("s")
    pltpu.sync_copy(z32, zbuf)
    if cnt_local is not None:
        @pl.loop(0, NPAD // 16)
        def _(i):
            cnt_local[pl.ds(i * 16, 16)] = jnp.zeros((16,), jnp.float32)

    def gstart(j):
        if j % 3 == 2:
            @pl.when(cid == 0)
            def _():
                pltpu.async_copy(tabs[0].at[sidx.at[j]], rows.at[j % NBUF],
                                 gsems.at[j % NBUF])

            @pl.when(cid == 1)
            def _():
                pltpu.async_copy(tabs[1].at[sidx.at[j]], rows.at[j % NBUF],
                                 gsems.at[j % NBUF])
        else:
            pltpu.async_copy(tab_sp.at[sidx.at[j]], rows.at[j % NBUF],
                             gsems.at[j % NBUF])

    def gwait(j):
        pltpu.make_async_copy(tabs[0].at[pl.ds(0, CW)], rows.at[j % NBUF],
                              gsems.at[j % NBUF]).wait()

    def sstart(j):
        pltpu.async_copy(rows.at[j % NBUF], acc.at[didx.at[j]],
                         ssems.at[j % NBUF], add=True)

    def swait(j):
        pltpu.make_async_copy(rows.at[j % NBUF], acc.at[pl.ds(0, CW)],
                              ssems.at[j % NBUF]).wait()

    for p in range(NP):
        q = p * NC + cid
        for r in range(RPT):
            row0 = (sid * RPT + r) * CW
            for qq in range(NQ):
                @pl.when(q == qq)
                def _():
                    pltpu.sync_copy(tabs[qq].at[pl.ds(row0, CW)], rows.at[0])
            pltpu.sync_copy(rows.at[0], tab_sp.at[pl.ds(row0, CW)])
            pltpu.sync_copy(zbuf, acc.at[pl.ds(row0, CW)])
        plsc.subcore_barrier()

        @pl.loop(0, NB)
        def _(b):
            base = sid * CPT + b * KB
            pltpu.sync_copy(srcr.at[pl.ds(base, KB)], sidx)
            pltpu.sync_copy(dstr.at[pl.ds(base, KB)], didx)
            for j in range(NBUF):
                gstart(j)
            for j in range(KB):
                gwait(j)
                sstart(j)
                k = j - LAG
                if k >= 0 and k + NBUF < KB:
                    swait(k)
                    gstart(k + NBUF)
            for j in range(KB - NBUF, KB):
                swait(j)

        plsc.subcore_barrier()
        for r in range(RPT):
            row0 = (sid * RPT + r) * CW
            pltpu.sync_copy(acc.at[pl.ds(row0, CW)], rows.at[0])
            pltpu.sync_copy(rows.at[0], oagg.at[q, pl.ds(row0, CW)])
        if p + 1 < NP:
            plsc.subcore_barrier()

    if cnt_local is not None:
        for b in range(NB // 2):
            base = sid * CPT + (cid * (NB // 2) + b) * KB
            pltpu.sync_copy(dstr.at[pl.ds(base, KB)], didx)

            @pl.loop(0, KB)
            def _(j):
                for k16 in range(CW // 16):
                    d = didx[j, pl.ds(k16 * 16, 16)]
                    cnts, last = plsc.scan_count(d)
                    plsc.addupdate_scatter(
                        cnt_local, [d], cnts.astype(jnp.float32), mask=last)

        wid = cid * NS + sid
        pltpu.sync_copy(cnt_local, ocnt.at[pl.ds(wid * NPAD, NPAD)])


def _sc_compiler_params():
    cp = pltpu.CompilerParams(use_tc_tiling_on_sc=False)
    if "needs_layout_passes" in pltpu.CompilerParams.__dataclass_fields__:
        cp = dataclasses.replace(cp, needs_layout_passes=False)
    return cp


def _make_sc(with_counts):
    mesh = plsc.VectorSubcoreMesh(core_axis_name="c", subcore_axis_name="s")
    agg_t = jax.ShapeDtypeStruct((NQ, NPAD, FW), jnp.float32)
    cnt_t = jax.ShapeDtypeStruct((NW * NPAD,), jnp.float32)
    scratch = [
        pltpu.VMEM((KB, CW), jnp.int32),
        pltpu.VMEM((KB, CW), jnp.int32),
        pltpu.VMEM((NBUF, CW, FW), jnp.float32),
        pltpu.VMEM((CW, FW), jnp.float32),
        pltpu.SemaphoreType.DMA((NBUF,)),
        pltpu.SemaphoreType.DMA((NBUF,)),
        pltpu.VMEM_SHARED((NPAD, FW), jnp.float32),
        pltpu.VMEM_SHARED((NPAD, FW), jnp.float32),
    ]
    if with_counts:
        scratch.append(pltpu.VMEM((NPAD,), jnp.float32))

        @functools.partial(pl.kernel, out_type=(agg_t, cnt_t), mesh=mesh,
                           scratch_types=scratch,
                           compiler_params=_sc_compiler_params())
        def k(t0, t1, srcr, dstr, z32, oagg, ocnt, sidx, didx,
              rows, zbuf, gsems, ssems, tab_sp, acc, cnt_local):
            _sc_body((t0, t1), srcr, dstr, z32, oagg, sidx, didx,
                     rows, zbuf, gsems, ssems, tab_sp, acc, ocnt=ocnt,
                     cnt_local=cnt_local)
    else:

        @functools.partial(pl.kernel, out_type=agg_t, mesh=mesh,
                           scratch_types=scratch,
                           compiler_params=_sc_compiler_params())
        def k(t0, t1, srcr, dstr, z32, oagg, sidx, didx,
              rows, zbuf, gsems, ssems, tab_sp, acc):
            _sc_body((t0, t1), srcr, dstr, z32, oagg, sidx, didx,
                     rows, zbuf, gsems, ssems, tab_sp, acc)

    return k


_sc_agg_counts = _make_sc(True)
_sc_agg_plain = _make_sc(False)


def _prep_call(x, eye, W1r, src2, dst2):
    def body(x_ref, e_ref, w_ref, s_ref, d_ref,
             t0_ref, t1_ref, r_ref, sp_ref, dp_ref):
        i = pl.program_id(0)
        xb = x_ref[...]
        t = lax.dot_general(xb, e_ref[...], (((0,), (0,)), ((), ())),
                            preferred_element_type=jnp.float32)
        t0_ref[...] = t[:, 0 * FW:1 * FW]
        t1_ref[...] = t[:, 1 * FW:2 * FW]
        r_ref[...] = lax.dot_general(t, w_ref[...], (((1,), (1,)), ((), ())),
                                     preferred_element_type=jnp.float32)
        rowid = i * EB + lax.broadcasted_iota(jnp.int32, (EB, CW), 0)
        valid = rowid < ER
        sp_ref[...] = jnp.where(valid, s_ref[...], 0)
        dp_ref[...] = jnp.where(valid, d_ref[...], N)

    tq_spec = pl.BlockSpec((BLK, FW), lambda i: (i, 0))
    tq_shape = jax.ShapeDtypeStruct((NPAD, FW), jnp.float32)
    return pl.pallas_call(
        body,
        grid=(NPAD // BLK,),
        in_specs=[pl.BlockSpec((D, BLK), lambda i: (0, i)),
                  pl.BlockSpec((D, D), lambda i: (0, 0)),
                  pl.BlockSpec((H, D), lambda i: (0, 0)),
                  pl.BlockSpec((EB, CW), lambda i: (i, 0)),
                  pl.BlockSpec((EB, CW), lambda i: (i, 0))],
        out_specs=[tq_spec, tq_spec,
                   pl.BlockSpec((BLK, H), lambda i: (i, 0)),
                   pl.BlockSpec((EB, CW), lambda i: (i, 0)),
                   pl.BlockSpec((EB, CW), lambda i: (i, 0))],
        out_shape=[tq_shape, tq_shape,
                   jax.ShapeDtypeStruct((NPAD, H), jnp.float32),
                   jax.ShapeDtypeStruct((ERP, CW), jnp.int32),
                   jax.ShapeDtypeStruct((ERP, CW), jnp.int32)],
        compiler_params=pltpu.CompilerParams(
            dimension_semantics=("parallel",)),
    )(x, eye, W1r, src2, dst2)


def _elu(v):
    return jnp.where(v > 0, v, jnp.exp(jnp.minimum(v, 0.0)) - 1.0)


def _inv_cnt(cv, ones_ref):
    cnt = lax.dot_general(cv, ones_ref[...], (((0,), (0,)), ((), ())),
                          preferred_element_type=jnp.float32)
    return 1.0 / jnp.maximum(cnt, 1.0)


def _mid_call(aggp, cntp, ones32, r1, W1l, b1, W2r):
    def body(a_ref, c_ref, o_ref, r_ref, wl_ref, b_ref, wn_ref,
             h0_ref, h1_ref, rn_ref, ic_ref):
        av = a_ref[...]
        a = jnp.concatenate([av[0], av[1]], axis=1)
        ic = _inv_cnt(c_ref[...], o_ref)
        ic_ref[...] = ic
        mean = a * ic
        v = lax.dot_general(mean, wl_ref[...], (((1,), (1,)), ((), ())),
                            preferred_element_type=jnp.float32)
        h = _elu(v + b_ref[...] + r_ref[...])
        h0_ref[...] = h[:, 0 * FW:1 * FW]
        h1_ref[...] = h[:, 1 * FW:2 * FW]
        rn_ref[...] = lax.dot_general(h, wn_ref[...], (((1,), (1,)), ((), ())),
                                      preferred_element_type=jnp.float32)

    hq_spec = pl.BlockSpec((BLK, FW), lambda i: (i, 0))
    hq_shape = jax.ShapeDtypeStruct((NPAD, FW), jnp.float32)
    return pl.pallas_call(
        body,
        grid=(NPAD // BLK,),
        in_specs=[pl.BlockSpec((NQ, BLK, FW), lambda i: (0, i, 0)),
                  pl.BlockSpec((NW, BLK), lambda i: (0, i)),
                  pl.BlockSpec((NW, 1), lambda i: (0, 0)),
                  pl.BlockSpec((BLK, H), lambda i: (i, 0)),
                  pl.BlockSpec((H, H), lambda i: (0, 0)),
                  pl.BlockSpec((1, H), lambda i: (0, 0)),
                  pl.BlockSpec((H, H), lambda i: (0, 0))],
        out_specs=[hq_spec, hq_spec,
                   pl.BlockSpec((BLK, H), lambda i: (i, 0)),
                   pl.BlockSpec((BLK, 1), lambda i: (i, 0))],
        out_shape=[hq_shape, hq_shape,
                   jax.ShapeDtypeStruct((NPAD, H), jnp.float32),
                   jax.ShapeDtypeStruct((NPAD, 1), jnp.float32)],
        compiler_params=pltpu.CompilerParams(
            dimension_semantics=("parallel",)),
    )(aggp, cntp, ones32, r1, W1l, b1, W2r)


def _final_call(aggp, icnt, r2, W2l, b2):
    def body(a_ref, ic_ref, r_ref, wl_ref, b_ref, out_ref):
        av = a_ref[...]
        a = jnp.concatenate([av[0], av[1]], axis=1)
        mean = a * ic_ref[...]
        v = lax.dot_general(mean, wl_ref[...], (((1,), (1,)), ((), ())),
                            preferred_element_type=jnp.float32)
        out_ref[...] = _elu(v + b_ref[...] + r_ref[...])

    return pl.pallas_call(
        body,
        grid=(N // FBLK,),
        in_specs=[pl.BlockSpec((NQ, FBLK, FW), lambda i: (0, i, 0)),
                  pl.BlockSpec((FBLK, 1), lambda i: (i, 0)),
                  pl.BlockSpec((FBLK, H), lambda i: (i, 0)),
                  pl.BlockSpec((H, H), lambda i: (0, 0)),
                  pl.BlockSpec((1, H), lambda i: (0, 0))],
        out_specs=pl.BlockSpec((FBLK, H), lambda i: (i, 0)),
        out_shape=jax.ShapeDtypeStruct((N, H), jnp.float32),
        compiler_params=pltpu.CompilerParams(
            dimension_semantics=("parallel",)),
    )(aggp, icnt, r2, W2l, b2)


def kernel(x, knn_edge_index, W1l, b1, W1r, W2l, b2, W2r):
    src2 = knn_edge_index[0].astype(jnp.int32).reshape(ER, CW)
    dst2 = knn_edge_index[1].astype(jnp.int32).reshape(ER, CW)
    eye = jnp.eye(D, dtype=jnp.float32)
    ones32 = jnp.ones((NW, 1), jnp.float32)
    z32 = jnp.zeros((CW, FW), jnp.float32)

    t0, t1, r1, srcp, dstp = _prep_call(x, eye, W1r, src2, dst2)
    agg1, cnt1 = _sc_agg_counts(t0, t1, srcp, dstp, z32)
    cnt1p = cnt1.reshape(NW, NPAD)
    h0, h1, r2, icnt = _mid_call(agg1, cnt1p, ones32, r1, W1l,
                                 b1.reshape(1, H), W2r)
    agg2 = _sc_agg_plain(h0, h1, srcp, dstp, z32)
    return _final_call(agg2, icnt, r2, W2l, b2.reshape(1, H))


# speedup vs baseline: 1.3333x; 1.3333x over previous
import dataclasses
import functools

import jax
import jax.numpy as jnp
from jax import lax
from jax.experimental import pallas as pl
from jax.experimental.pallas import tpu as pltpu
from jax.experimental.pallas import tpu_sc as plsc

N = 10000
D = 128
H = 128
E = 320000

NC = 2
NS = 16
NW = NC * NS

FW = 64
NQ = 2
NP = NQ // NC
CW = 128
CPT = 160
EPAD = NS * CPT * CW
KB = 16
NB = CPT // KB
NBUF = 4
LAG = 2
NPAD = 10240
RPT = NPAD // (NS * CW)

BLK = 512
ER = E // CW
ERP = NS * CPT
EB = ERP // (NPAD // BLK)
FBLK = 400


def _sc_body(tabs, srcr, dstr, z32, oagg, sidx, didx, rows, gsems,
             ssems, tab_sp, acc, ocnt=None, cnt_local=None):
    cid = lax.axis_index("c")
    sid = lax.axis_index("s")
    if cnt_local is not None:
        @pl.loop(0, NPAD // 16)
        def _(i):
            cnt_local[pl.ds(i * 16, 16)] = jnp.zeros((16,), jnp.float32)

    def gstart(j):
        pltpu.async_copy(tab_sp.at[sidx.at[j]], rows.at[j % NBUF],
                         gsems.at[j % NBUF])

    def gwait(j):
        pltpu.make_async_copy(tabs[0].at[pl.ds(0, CW)], rows.at[j % NBUF],
                              gsems.at[j % NBUF]).wait()

    def sstart(j):
        pltpu.async_copy(rows.at[j % NBUF], acc.at[didx.at[j]],
                         ssems.at[j % NBUF], add=True)

    def swait(j):
        pltpu.make_async_copy(rows.at[j % NBUF], acc.at[pl.ds(0, CW)],
                              ssems.at[j % NBUF]).wait()

    for p in range(NP):
        q = p * NC + cid
        pltpu.sync_copy(z32, rows.at[1])
        for r in range(RPT):
            row0 = (sid * RPT + r) * CW
            for qq in range(NQ):
                @pl.when(q == qq)
                def _():
                    pltpu.sync_copy(tabs[qq].at[pl.ds(row0, CW)], rows.at[0])
            pltpu.sync_copy(rows.at[0], tab_sp.at[pl.ds(row0, CW)])
            pltpu.sync_copy(rows.at[1], acc.at[pl.ds(row0, CW)])
        plsc.subcore_barrier()

        @pl.loop(0, NB)
        def _(b):
            base = sid * CPT + b * KB
            pltpu.sync_copy(srcr.at[pl.ds(base, KB)], sidx)
            pltpu.sync_copy(dstr.at[pl.ds(base, KB)], didx)
            for j in range(NBUF):
                gstart(j)
            for j in range(KB):
                gwait(j)
                sstart(j)
                k = j - LAG
                if k >= 0 and k + NBUF < KB:
                    swait(k)
                    gstart(k + NBUF)
            for j in range(KB - NBUF, KB):
                swait(j)

        plsc.subcore_barrier()
        for r in range(RPT):
            row0 = (sid * RPT + r) * CW
            pltpu.sync_copy(acc.at[pl.ds(row0, CW)], rows.at[0])
            pltpu.sync_copy(rows.at[0], oagg.at[q, pl.ds(row0, CW)])
        if p + 1 < NP:
            plsc.subcore_barrier()

    if cnt_local is not None:
        for b in range(NB // 2):
            base = sid * CPT + (cid * (NB // 2) + b) * KB
            pltpu.sync_copy(dstr.at[pl.ds(base, KB)], didx)

            @pl.loop(0, KB)
            def _(j):
                for k16 in range(CW // 16):
                    d = didx[j, pl.ds(k16 * 16, 16)]
                    cnts, last = plsc.scan_count(d)
                    plsc.addupdate_scatter(
                        cnt_local, [d], cnts.astype(jnp.float32), mask=last)

        wid = cid * NS + sid
        pltpu.sync_copy(cnt_local, ocnt.at[pl.ds(wid * NPAD, NPAD)])


def _sc_compiler_params():
    cp = pltpu.CompilerParams(use_tc_tiling_on_sc=False)
    if "needs_layout_passes" in pltpu.CompilerParams.__dataclass_fields__:
        cp = dataclasses.replace(cp, needs_layout_passes=False)
    return cp


def _make_sc(with_counts):
    mesh = plsc.VectorSubcoreMesh(core_axis_name="c", subcore_axis_name="s")
    agg_t = jax.ShapeDtypeStruct((NQ, NPAD, FW), jnp.float32)
    cnt_t = jax.ShapeDtypeStruct((NW * NPAD,), jnp.float32)
    scratch = [
        pltpu.VMEM((KB, CW), jnp.int32),
        pltpu.VMEM((KB, CW), jnp.int32),
        pltpu.VMEM((NBUF, CW, FW), jnp.float32),
        pltpu.SemaphoreType.DMA((NBUF,)),
        pltpu.SemaphoreType.DMA((NBUF,)),
        pltpu.VMEM_SHARED((NPAD, FW), jnp.float32),
        pltpu.VMEM_SHARED((NPAD, FW), jnp.float32),
    ]
    if with_counts:
        scratch.append(pltpu.VMEM((NPAD,), jnp.float32))

        @functools.partial(pl.kernel, out_type=(agg_t, cnt_t), mesh=mesh,
                           scratch_types=scratch,
                           compiler_params=_sc_compiler_params())
        def k(t0, t1, srcr, dstr, z32, oagg, ocnt, sidx, didx,
              rows, gsems, ssems, tab_sp, acc, cnt_local):
            _sc_body((t0, t1), srcr, dstr, z32, oagg, sidx, didx,
                     rows, gsems, ssems, tab_sp, acc, ocnt=ocnt,
                     cnt_local=cnt_local)
    else:

        @functools.partial(pl.kernel, out_type=agg_t, mesh=mesh,
                           scratch_types=scratch,
                           compiler_params=_sc_compiler_params())
        def k(t0, t1, srcr, dstr, z32, oagg, sidx, didx,
              rows, gsems, ssems, tab_sp, acc):
            _sc_body((t0, t1), srcr, dstr, z32, oagg, sidx, didx,
                     rows, gsems, ssems, tab_sp, acc)

    return k


_sc_agg_counts = _make_sc(True)
_sc_agg_plain = _make_sc(False)


def _prep_call(x, eye, W1r, src2, dst2):
    def body(x_ref, e_ref, w_ref, s_ref, d_ref,
             t0_ref, t1_ref, r_ref, sp_ref, dp_ref):
        i = pl.program_id(0)
        xb = x_ref[...]
        t = lax.dot_general(xb, e_ref[...], (((0,), (0,)), ((), ())),
                            preferred_element_type=jnp.float32)
        t0_ref[...] = t[:, 0 * FW:1 * FW]
        t1_ref[...] = t[:, 1 * FW:2 * FW]
        r_ref[...] = lax.dot_general(t, w_ref[...], (((1,), (1,)), ((), ())),
                                     preferred_element_type=jnp.float32)
        rowid = i * EB + lax.broadcasted_iota(jnp.int32, (EB, CW), 0)
        valid = rowid < ER
        sp_ref[...] = jnp.where(valid, s_ref[...], 0)
        dp_ref[...] = jnp.where(valid, d_ref[...], N)

    tq_spec = pl.BlockSpec((BLK, FW), lambda i: (i, 0))
    tq_shape = jax.ShapeDtypeStruct((NPAD, FW), jnp.float32)
    return pl.pallas_call(
        body,
        grid=(NPAD // BLK,),
        in_specs=[pl.BlockSpec((D, BLK), lambda i: (0, i)),
                  pl.BlockSpec((D, D), lambda i: (0, 0)),
                  pl.BlockSpec((H, D), lambda i: (0, 0)),
                  pl.BlockSpec((EB, CW), lambda i: (i, 0)),
                  pl.BlockSpec((EB, CW), lambda i: (i, 0))],
        out_specs=[tq_spec, tq_spec,
                   pl.BlockSpec((BLK, H), lambda i: (i, 0)),
                   pl.BlockSpec((EB, CW), lambda i: (i, 0)),
                   pl.BlockSpec((EB, CW), lambda i: (i, 0))],
        out_shape=[tq_shape, tq_shape,
                   jax.ShapeDtypeStruct((NPAD, H), jnp.float32),
                   jax.ShapeDtypeStruct((ERP, CW), jnp.int32),
                   jax.ShapeDtypeStruct((ERP, CW), jnp.int32)],
        compiler_params=pltpu.CompilerParams(
            dimension_semantics=("parallel",)),
    )(x, eye, W1r, src2, dst2)


def _elu(v):
    return jnp.where(v > 0, v, jnp.exp(jnp.minimum(v, 0.0)) - 1.0)


def _inv_cnt(cv, ones_ref):
    cnt = lax.dot_general(cv, ones_ref[...], (((0,), (0,)), ((), ())),
                          preferred_element_type=jnp.float32)
    return 1.0 / jnp.maximum(cnt, 1.0)


def _mid_call(aggp, cntp, ones32, r1, W1l, b1, W2r):
    def body(a_ref, c_ref, o_ref, r_ref, wl_ref, b_ref, wn_ref,
             h0_ref, h1_ref, rn_ref, ic_ref):
        av = a_ref[...]
        a = jnp.concatenate([av[0], av[1]], axis=1)
        ic = _inv_cnt(c_ref[...], o_ref)
        ic_ref[...] = ic
        mean = a * ic
        v = lax.dot_general(mean, wl_ref[...], (((1,), (1,)), ((), ())),
                            preferred_element_type=jnp.float32)
        h = _elu(v + b_ref[...] + r_ref[...])
        h0_ref[...] = h[:, 0 * FW:1 * FW]
        h1_ref[...] = h[:, 1 * FW:2 * FW]
        rn_ref[...] = lax.dot_general(h, wn_ref[...], (((1,), (1,)), ((), ())),
                                      preferred_element_type=jnp.float32)

    hq_spec = pl.BlockSpec((BLK, FW), lambda i: (i, 0))
    hq_shape = jax.ShapeDtypeStruct((NPAD, FW), jnp.float32)
    return pl.pallas_call(
        body,
        grid=(NPAD // BLK,),
        in_specs=[pl.BlockSpec((NQ, BLK, FW), lambda i: (0, i, 0)),
                  pl.BlockSpec((NW, BLK), lambda i: (0, i)),
                  pl.BlockSpec((NW, 1), lambda i: (0, 0)),
                  pl.BlockSpec((BLK, H), lambda i: (i, 0)),
                  pl.BlockSpec((H, H), lambda i: (0, 0)),
                  pl.BlockSpec((1, H), lambda i: (0, 0)),
                  pl.BlockSpec((H, H), lambda i: (0, 0))],
        out_specs=[hq_spec, hq_spec,
                   pl.BlockSpec((BLK, H), lambda i: (i, 0)),
                   pl.BlockSpec((BLK, 1), lambda i: (i, 0))],
        out_shape=[hq_shape, hq_shape,
                   jax.ShapeDtypeStruct((NPAD, H), jnp.float32),
                   jax.ShapeDtypeStruct((NPAD, 1), jnp.float32)],
        compiler_params=pltpu.CompilerParams(
            dimension_semantics=("parallel",)),
    )(aggp, cntp, ones32, r1, W1l, b1, W2r)


def _final_call(aggp, icnt, r2, W2l, b2):
    def body(a_ref, ic_ref, r_ref, wl_ref, b_ref, out_ref):
        av = a_ref[...]
        a = jnp.concatenate([av[0], av[1]], axis=1)
        mean = a * ic_ref[...]
        v = lax.dot_general(mean, wl_ref[...], (((1,), (1,)), ((), ())),
                            preferred_element_type=jnp.float32)
        out_ref[...] = _elu(v + b_ref[...] + r_ref[...])

    return pl.pallas_call(
        body,
        grid=(N // FBLK,),
        in_specs=[pl.BlockSpec((NQ, FBLK, FW), lambda i: (0, i, 0)),
                  pl.BlockSpec((FBLK, 1), lambda i: (i, 0)),
                  pl.BlockSpec((FBLK, H), lambda i: (i, 0)),
                  pl.BlockSpec((H, H), lambda i: (0, 0)),
                  pl.BlockSpec((1, H), lambda i: (0, 0))],
        out_specs=pl.BlockSpec((FBLK, H), lambda i: (i, 0)),
        out_shape=jax.ShapeDtypeStruct((N, H), jnp.float32),
        compiler_params=pltpu.CompilerParams(
            dimension_semantics=("parallel",)),
    )(aggp, icnt, r2, W2l, b2)


def kernel(x, knn_edge_index, W1l, b1, W1r, W2l, b2, W2r):
    src2 = knn_edge_index[0].astype(jnp.int32).reshape(ER, CW)
    dst2 = knn_edge_index[1].astype(jnp.int32).reshape(ER, CW)
    eye = jnp.eye(D, dtype=jnp.float32)
    ones32 = jnp.ones((NW, 1), jnp.float32)
    z32 = jnp.zeros((CW, FW), jnp.float32)

    t0, t1, r1, srcp, dstp = _prep_call(x, eye, W1r, src2, dst2)
    agg1, cnt1 = _sc_agg_counts(t0, t1, srcp, dstp, z32)
    cnt1p = cnt1.reshape(NW, NPAD)
    h0, h1, r2, icnt = _mid_call(agg1, cnt1p, ones32, r1, W1l,
                                 b1.reshape(1, H), W2r)
    agg2 = _sc_agg_plain(h0, h1, srcp, dstp, z32)
    return _final_call(agg2, icnt, r2, W2l, b2.reshape(1, H))


# speedup vs baseline: 1.3379x; 1.0035x over previous
import dataclasses
import functools

import jax
import jax.numpy as jnp
from jax import lax
from jax.experimental import pallas as pl
from jax.experimental.pallas import tpu as pltpu
from jax.experimental.pallas import tpu_sc as plsc

N = 10000
D = 128
H = 128
E = 320000

NC = 2
NS = 16
NW = NC * NS

FW = 64
NQ = 2
NP = NQ // NC
CW = 128
CPT = 160
EPAD = NS * CPT * CW
KB = 16
NB = CPT // KB
NBUF = 4
LAG = 1
NPAD = 10240
RPT = NPAD // (NS * CW)

BLK = 512
ER = E // CW
ERP = NS * CPT
EB = ERP // (NPAD // BLK)
FBLK = 400


def _sc_body(tabs, srcr, dstr, z32, oagg, sidx, didx, rows, gsems,
             ssems, tab_sp, acc, ocnt=None, cnt_local=None):
    cid = lax.axis_index("c")
    sid = lax.axis_index("s")
    if cnt_local is not None:
        @pl.loop(0, NPAD // 16)
        def _(i):
            cnt_local[pl.ds(i * 16, 16)] = jnp.zeros((16,), jnp.float32)

    def gstart(j):
        pltpu.async_copy(tab_sp.at[sidx.at[j]], rows.at[j % NBUF],
                         gsems.at[j % NBUF])

    def gwait(j):
        pltpu.make_async_copy(tabs[0].at[pl.ds(0, CW)], rows.at[j % NBUF],
                              gsems.at[j % NBUF]).wait()

    def sstart(j):
        pltpu.async_copy(rows.at[j % NBUF], acc.at[didx.at[j]],
                         ssems.at[j % NBUF], add=True)

    def swait(j):
        pltpu.make_async_copy(rows.at[j % NBUF], acc.at[pl.ds(0, CW)],
                              ssems.at[j % NBUF]).wait()

    for p in range(NP):
        q = p * NC + cid
        pltpu.sync_copy(z32, rows.at[1])
        for r in range(RPT):
            row0 = (sid * RPT + r) * CW
            for qq in range(NQ):
                @pl.when(q == qq)
                def _():
                    pltpu.sync_copy(tabs[qq].at[pl.ds(row0, CW)], rows.at[0])
            pltpu.sync_copy(rows.at[0], tab_sp.at[pl.ds(row0, CW)])
            pltpu.sync_copy(rows.at[1], acc.at[pl.ds(row0, CW)])
        plsc.subcore_barrier()

        @pl.loop(0, NB)
        def _(b):
            base = sid * CPT + b * KB
            pltpu.sync_copy(srcr.at[pl.ds(base, KB)], sidx)
            pltpu.sync_copy(dstr.at[pl.ds(base, KB)], didx)
            for j in range(NBUF):
                gstart(j)
            for j in range(KB):
                gwait(j)
                sstart(j)
                k = j - LAG
                if k >= 0 and k + NBUF < KB:
                    swait(k)
                    gstart(k + NBUF)
            for j in range(KB - NBUF, KB):
                swait(j)

        plsc.subcore_barrier()
        for r in range(RPT):
            row0 = (sid * RPT + r) * CW
            pltpu.sync_copy(acc.at[pl.ds(row0, CW)], rows.at[0])
            pltpu.sync_copy(rows.at[0], oagg.at[q, pl.ds(row0, CW)])
        if p + 1 < NP:
            plsc.subcore_barrier()

    if cnt_local is not None:
        for b in range(NB // 2):
            base = sid * CPT + (cid * (NB // 2) + b) * KB
            pltpu.sync_copy(dstr.at[pl.ds(base, KB)], didx)

            @pl.loop(0, KB)
            def _(j):
                for k16 in range(CW // 16):
                    d = didx[j, pl.ds(k16 * 16, 16)]
                    cnts, last = plsc.scan_count(d)
                    plsc.addupdate_scatter(
                        cnt_local, [d], cnts.astype(jnp.float32), mask=last)

        wid = cid * NS + sid
        pltpu.sync_copy(cnt_local, ocnt.at[pl.ds(wid * NPAD, NPAD)])


def _sc_compiler_params():
    cp = pltpu.CompilerParams(use_tc_tiling_on_sc=False)
    if "needs_layout_passes" in pltpu.CompilerParams.__dataclass_fields__:
        cp = dataclasses.replace(cp, needs_layout_passes=False)
    return cp


def _make_sc(with_counts):
    mesh = plsc.VectorSubcoreMesh(core_axis_name="c", subcore_axis_name="s")
    agg_t = jax.ShapeDtypeStruct((NQ, NPAD, FW), jnp.float32)
    cnt_t = jax.ShapeDtypeStruct((NW * NPAD,), jnp.float32)
    scratch = [
        pltpu.VMEM((KB, CW), jnp.int32),
        pltpu.VMEM((KB, CW), jnp.int32),
        pltpu.VMEM((NBUF, CW, FW), jnp.float32),
        pltpu.SemaphoreType.DMA((NBUF,)),
        pltpu.SemaphoreType.DMA((NBUF,)),
        pltpu.VMEM_SHARED((NPAD, FW), jnp.float32),
        pltpu.VMEM_SHARED((NPAD, FW), jnp.float32),
    ]
    if with_counts:
        scratch.append(pltpu.VMEM((NPAD,), jnp.float32))

        @functools.partial(pl.kernel, out_type=(agg_t, cnt_t), mesh=mesh,
                           scratch_types=scratch,
                           compiler_params=_sc_compiler_params())
        def k(t0, t1, srcr, dstr, z32, oagg, ocnt, sidx, didx,
              rows, gsems, ssems, tab_sp, acc, cnt_local):
            _sc_body((t0, t1), srcr, dstr, z32, oagg, sidx, didx,
                     rows, gsems, ssems, tab_sp, acc, ocnt=ocnt,
                     cnt_local=cnt_local)
    else:

        @functools.partial(pl.kernel, out_type=agg_t, mesh=mesh,
                           scratch_types=scratch,
                           compiler_params=_sc_compiler_params())
        def k(t0, t1, srcr, dstr, z32, oagg, sidx, didx,
              rows, gsems, ssems, tab_sp, acc):
            _sc_body((t0, t1), srcr, dstr, z32, oagg, sidx, didx,
                     rows, gsems, ssems, tab_sp, acc)

    return k


_sc_agg_counts = _make_sc(True)
_sc_agg_plain = _make_sc(False)


def _prep_call(x, eye, W1r, src2, dst2):
    def body(x_ref, e_ref, w_ref, s_ref, d_ref,
             t0_ref, t1_ref, r_ref, sp_ref, dp_ref):
        i = pl.program_id(0)
        xb = x_ref[...]
        t = lax.dot_general(xb, e_ref[...], (((0,), (0,)), ((), ())),
                            preferred_element_type=jnp.float32)
        t0_ref[...] = t[:, 0 * FW:1 * FW]
        t1_ref[...] = t[:, 1 * FW:2 * FW]
        r_ref[...] = lax.dot_general(t, w_ref[...], (((1,), (1,)), ((), ())),
                                     preferred_element_type=jnp.float32)
        rowid = i * EB + lax.broadcasted_iota(jnp.int32, (EB, CW), 0)
        valid = rowid < ER
        sp_ref[...] = jnp.where(valid, s_ref[...], 0)
        dp_ref[...] = jnp.where(valid, d_ref[...], N)

    tq_spec = pl.BlockSpec((BLK, FW), lambda i: (i, 0))
    tq_shape = jax.ShapeDtypeStruct((NPAD, FW), jnp.float32)
    return pl.pallas_call(
        body,
        grid=(NPAD // BLK,),
        in_specs=[pl.BlockSpec((D, BLK), lambda i: (0, i)),
                  pl.BlockSpec((D, D), lambda i: (0, 0)),
                  pl.BlockSpec((H, D), lambda i: (0, 0)),
                  pl.BlockSpec((EB, CW), lambda i: (i, 0)),
                  pl.BlockSpec((EB, CW), lambda i: (i, 0))],
        out_specs=[tq_spec, tq_spec,
                   pl.BlockSpec((BLK, H), lambda i: (i, 0)),
                   pl.BlockSpec((EB, CW), lambda i: (i, 0)),
                   pl.BlockSpec((EB, CW), lambda i: (i, 0))],
        out_shape=[tq_shape, tq_shape,
                   jax.ShapeDtypeStruct((NPAD, H), jnp.float32),
                   jax.ShapeDtypeStruct((ERP, CW), jnp.int32),
                   jax.ShapeDtypeStruct((ERP, CW), jnp.int32)],
        compiler_params=pltpu.CompilerParams(
            dimension_semantics=("parallel",)),
    )(x, eye, W1r, src2, dst2)


def _elu(v):
    return jnp.where(v > 0, v, jnp.exp(jnp.minimum(v, 0.0)) - 1.0)


def _inv_cnt(cv, ones_ref):
    cnt = lax.dot_general(cv, ones_ref[...], (((0,), (0,)), ((), ())),
                          preferred_element_type=jnp.float32)
    return 1.0 / jnp.maximum(cnt, 1.0)


def _mid_call(aggp, cntp, ones32, r1, W1l, b1, W2r):
    def body(a_ref, c_ref, o_ref, r_ref, wl_ref, b_ref, wn_ref,
             h0_ref, h1_ref, rn_ref, ic_ref):
        av = a_ref[...]
        a = jnp.concatenate([av[0], av[1]], axis=1)
        ic = _inv_cnt(c_ref[...], o_ref)
        ic_ref[...] = ic
        mean = a * ic
        v = lax.dot_general(mean, wl_ref[...], (((1,), (1,)), ((), ())),
                            preferred_element_type=jnp.float32)
        h = _elu(v + b_ref[...] + r_ref[...])
        h0_ref[...] = h[:, 0 * FW:1 * FW]
        h1_ref[...] = h[:, 1 * FW:2 * FW]
        rn_ref[...] = lax.dot_general(h, wn_ref[...], (((1,), (1,)), ((), ())),
                                      preferred_element_type=jnp.float32)

    hq_spec = pl.BlockSpec((BLK, FW), lambda i: (i, 0))
    hq_shape = jax.ShapeDtypeStruct((NPAD, FW), jnp.float32)
    return pl.pallas_call(
        body,
        grid=(NPAD // BLK,),
        in_specs=[pl.BlockSpec((NQ, BLK, FW), lambda i: (0, i, 0)),
                  pl.BlockSpec((NW, BLK), lambda i: (0, i)),
                  pl.BlockSpec((NW, 1), lambda i: (0, 0)),
                  pl.BlockSpec((BLK, H), lambda i: (i, 0)),
                  pl.BlockSpec((H, H), lambda i: (0, 0)),
                  pl.BlockSpec((1, H), lambda i: (0, 0)),
                  pl.BlockSpec((H, H), lambda i: (0, 0))],
        out_specs=[hq_spec, hq_spec,
                   pl.BlockSpec((BLK, H), lambda i: (i, 0)),
                   pl.BlockSpec((BLK, 1), lambda i: (i, 0))],
        out_shape=[hq_shape, hq_shape,
                   jax.ShapeDtypeStruct((NPAD, H), jnp.float32),
                   jax.ShapeDtypeStruct((NPAD, 1), jnp.float32)],
        compiler_params=pltpu.CompilerParams(
            dimension_semantics=("parallel",)),
    )(aggp, cntp, ones32, r1, W1l, b1, W2r)


def _final_call(aggp, icnt, r2, W2l, b2):
    def body(a_ref, ic_ref, r_ref, wl_ref, b_ref, out_ref):
        av = a_ref[...]
        a = jnp.concatenate([av[0], av[1]], axis=1)
        mean = a * ic_ref[...]
        v = lax.dot_general(mean, wl_ref[...], (((1,), (1,)), ((), ())),
                            preferred_element_type=jnp.float32)
        out_ref[...] = _elu(v + b_ref[...] + r_ref[...])

    return pl.pallas_call(
        body,
        grid=(N // FBLK,),
        in_specs=[pl.BlockSpec((NQ, FBLK, FW), lambda i: (0, i, 0)),
                  pl.BlockSpec((FBLK, 1), lambda i: (i, 0)),
                  pl.BlockSpec((FBLK, H), lambda i: (i, 0)),
                  pl.BlockSpec((H, H), lambda i: (0, 0)),
                  pl.BlockSpec((1, H), lambda i: (0, 0))],
        out_specs=pl.BlockSpec((FBLK, H), lambda i: (i, 0)),
        out_shape=jax.ShapeDtypeStruct((N, H), jnp.float32),
        compiler_params=pltpu.CompilerParams(
            dimension_semantics=("parallel",)),
    )(aggp, icnt, r2, W2l, b2)


def kernel(x, knn_edge_index, W1l, b1, W1r, W2l, b2, W2r):
    src2 = knn_edge_index[0].astype(jnp.int32).reshape(ER, CW)
    dst2 = knn_edge_index[1].astype(jnp.int32).reshape(ER, CW)
    eye = jnp.eye(D, dtype=jnp.float32)
    ones32 = jnp.ones((NW, 1), jnp.float32)
    z32 = jnp.zeros((CW, FW), jnp.float32)

    t0, t1, r1, srcp, dstp = _prep_call(x, eye, W1r, src2, dst2)
    agg1, cnt1 = _sc_agg_counts(t0, t1, srcp, dstp, z32)
    cnt1p = cnt1.reshape(NW, NPAD)
    h0, h1, r2, icnt = _mid_call(agg1, cnt1p, ones32, r1, W1l,
                                 b1.reshape(1, H), W2r)
    agg2 = _sc_agg_plain(h0, h1, srcp, dstp, z32)
    return _final_call(agg2, icnt, r2, W2l, b2.reshape(1, H))


# speedup vs baseline: 1.3786x; 1.0304x over previous
import dataclasses
import functools

import jax
import jax.numpy as jnp
from jax import lax
from jax.experimental import pallas as pl
from jax.experimental.pallas import tpu as pltpu
from jax.experimental.pallas import tpu_sc as plsc

N = 10000
D = 128
H = 128
E = 320000

NC = 2
NS = 16
NW = NC * NS

FW = 64
NQ = 2
NP = NQ // NC
CW = 128
CPT = 160
EPAD = NS * CPT * CW
KB = 20
NB = CPT // KB
NBUF = 4
LAG = 1
NPAD = 10240
RPT = NPAD // (NS * CW)

BLK = 512
ER = E // CW
ERP = NS * CPT
EB = ERP // (NPAD // BLK)
FBLK = 400


def _sc_body(tabs, srcr, dstr, z32, oagg, sidx, didx, rows, gsems,
             ssems, tab_sp, acc, ocnt=None, cnt_local=None):
    cid = lax.axis_index("c")
    sid = lax.axis_index("s")
    if cnt_local is not None:
        @pl.loop(0, NPAD // 16)
        def _(i):
            cnt_local[pl.ds(i * 16, 16)] = jnp.zeros((16,), jnp.float32)

    def gstart(j):
        pltpu.async_copy(tab_sp.at[sidx.at[j]], rows.at[j % NBUF],
                         gsems.at[j % NBUF])

    def gwait(j):
        pltpu.make_async_copy(tabs[0].at[pl.ds(0, CW)], rows.at[j % NBUF],
                              gsems.at[j % NBUF]).wait()

    def sstart(j):
        pltpu.async_copy(rows.at[j % NBUF], acc.at[didx.at[j]],
                         ssems.at[j % NBUF], add=True)

    def swait(j):
        pltpu.make_async_copy(rows.at[j % NBUF], acc.at[pl.ds(0, CW)],
                              ssems.at[j % NBUF]).wait()

    for p in range(NP):
        q = p * NC + cid
        pltpu.sync_copy(z32, rows.at[1])
        for r in range(RPT):
            row0 = (sid * RPT + r) * CW
            for qq in range(NQ):
                @pl.when(q == qq)
                def _():
                    pltpu.sync_copy(tabs[qq].at[pl.ds(row0, CW)], rows.at[0])
            pltpu.sync_copy(rows.at[0], tab_sp.at[pl.ds(row0, CW)])
            pltpu.sync_copy(rows.at[1], acc.at[pl.ds(row0, CW)])
        plsc.subcore_barrier()

        @pl.loop(0, NB)
        def _(b):
            base = sid * CPT + b * KB
            pltpu.sync_copy(srcr.at[pl.ds(base, KB)], sidx)
            pltpu.sync_copy(dstr.at[pl.ds(base, KB)], didx)
            for j in range(NBUF):
                gstart(j)
            for j in range(KB):
                gwait(j)
                sstart(j)
                k = j - LAG
                if k >= 0 and k + NBUF < KB:
                    swait(k)
                    gstart(k + NBUF)
            for j in range(KB - NBUF, KB):
                swait(j)

        plsc.subcore_barrier()
        for r in range(RPT):
            row0 = (sid * RPT + r) * CW
            pltpu.sync_copy(acc.at[pl.ds(row0, CW)], rows.at[0])
            pltpu.sync_copy(rows.at[0], oagg.at[q, pl.ds(row0, CW)])
        if p + 1 < NP:
            plsc.subcore_barrier()

    if cnt_local is not None:
        for b in range(NB // 2):
            base = sid * CPT + (cid * (NB // 2) + b) * KB
            pltpu.sync_copy(dstr.at[pl.ds(base, KB)], didx)

            @pl.loop(0, KB)
            def _(j):
                for k16 in range(CW // 16):
                    d = didx[j, pl.ds(k16 * 16, 16)]
                    cnts, last = plsc.scan_count(d)
                    plsc.addupdate_scatter(
                        cnt_local, [d], cnts.astype(jnp.float32), mask=last)

        wid = cid * NS + sid
        pltpu.sync_copy(cnt_local, ocnt.at[pl.ds(wid * NPAD, NPAD)])


def _sc_compiler_params():
    cp = pltpu.CompilerParams(use_tc_tiling_on_sc=False)
    if "needs_layout_passes" in pltpu.CompilerParams.__dataclass_fields__:
        cp = dataclasses.replace(cp, needs_layout_passes=False)
    return cp


def _make_sc(with_counts):
    mesh = plsc.VectorSubcoreMesh(core_axis_name="c", subcore_axis_name="s")
    agg_t = jax.ShapeDtypeStruct((NQ, NPAD, FW), jnp.float32)
    cnt_t = jax.ShapeDtypeStruct((NW * NPAD,), jnp.float32)
    scratch = [
        pltpu.VMEM((KB, CW), jnp.int32),
        pltpu.VMEM((KB, CW), jnp.int32),
        pltpu.VMEM((NBUF, CW, FW), jnp.float32),
        pltpu.SemaphoreType.DMA((NBUF,)),
        pltpu.SemaphoreType.DMA((NBUF,)),
        pltpu.VMEM_SHARED((NPAD, FW), jnp.float32),
        pltpu.VMEM_SHARED((NPAD, FW), jnp.float32),
    ]
    if with_counts:
        scratch.append(pltpu.VMEM((NPAD,), jnp.float32))

        @functools.partial(pl.kernel, out_type=(agg_t, cnt_t), mesh=mesh,
                           scratch_types=scratch,
                           compiler_params=_sc_compiler_params())
        def k(t0, t1, srcr, dstr, z32, oagg, ocnt, sidx, didx,
              rows, gsems, ssems, tab_sp, acc, cnt_local):
            _sc_body((t0, t1), srcr, dstr, z32, oagg, sidx, didx,
                     rows, gsems, ssems, tab_sp, acc, ocnt=ocnt,
                     cnt_local=cnt_local)
    else:

        @functools.partial(pl.kernel, out_type=agg_t, mesh=mesh,
                           scratch_types=scratch,
                           compiler_params=_sc_compiler_params())
        def k(t0, t1, srcr, dstr, z32, oagg, sidx, didx,
              rows, gsems, ssems, tab_sp, acc):
            _sc_body((t0, t1), srcr, dstr, z32, oagg, sidx, didx,
                     rows, gsems, ssems, tab_sp, acc)

    return k


_sc_agg_counts = _make_sc(True)
_sc_agg_plain = _make_sc(False)


def _prep_call(x, eye, W1r, src2, dst2):
    def body(x_ref, e_ref, w_ref, s_ref, d_ref,
             t0_ref, t1_ref, r_ref, sp_ref, dp_ref):
        i = pl.program_id(0)
        xb = x_ref[...]
        t = lax.dot_general(xb, e_ref[...], (((0,), (0,)), ((), ())),
                            preferred_element_type=jnp.float32)
        t0_ref[...] = t[:, 0 * FW:1 * FW]
        t1_ref[...] = t[:, 1 * FW:2 * FW]
        r_ref[...] = lax.dot_general(t, w_ref[...], (((1,), (1,)), ((), ())),
                                     preferred_element_type=jnp.float32)
        rowid = i * EB + lax.broadcasted_iota(jnp.int32, (EB, CW), 0)
        valid = rowid < ER
        sp_ref[...] = jnp.where(valid, s_ref[...], 0)
        dp_ref[...] = jnp.where(valid, d_ref[...], N)

    tq_spec = pl.BlockSpec((BLK, FW), lambda i: (i, 0))
    tq_shape = jax.ShapeDtypeStruct((NPAD, FW), jnp.float32)
    return pl.pallas_call(
        body,
        grid=(NPAD // BLK,),
        in_specs=[pl.BlockSpec((D, BLK), lambda i: (0, i)),
                  pl.BlockSpec((D, D), lambda i: (0, 0)),
                  pl.BlockSpec((H, D), lambda i: (0, 0)),
                  pl.BlockSpec((EB, CW), lambda i: (i, 0)),
                  pl.BlockSpec((EB, CW), lambda i: (i, 0))],
        out_specs=[tq_spec, tq_spec,
                   pl.BlockSpec((BLK, H), lambda i: (i, 0)),
                   pl.BlockSpec((EB, CW), lambda i: (i, 0)),
                   pl.BlockSpec((EB, CW), lambda i: (i, 0))],
        out_shape=[tq_shape, tq_shape,
                   jax.ShapeDtypeStruct((NPAD, H), jnp.float32),
                   jax.ShapeDtypeStruct((ERP, CW), jnp.int32),
                   jax.ShapeDtypeStruct((ERP, CW), jnp.int32)],
        compiler_params=pltpu.CompilerParams(
            dimension_semantics=("parallel",)),
    )(x, eye, W1r, src2, dst2)


def _elu(v):
    return jnp.where(v > 0, v, jnp.exp(jnp.minimum(v, 0.0)) - 1.0)


def _inv_cnt(cv, ones_ref):
    cnt = lax.dot_general(cv, ones_ref[...], (((0,), (0,)), ((), ())),
                          preferred_element_type=jnp.float32)
    return 1.0 / jnp.maximum(cnt, 1.0)


def _mid_call(aggp, cntp, ones32, r1, W1l, b1, W2r):
    def body(a_ref, c_ref, o_ref, r_ref, wl_ref, b_ref, wn_ref,
             h0_ref, h1_ref, rn_ref, ic_ref):
        av = a_ref[...]
        a = jnp.concatenate([av[0], av[1]], axis=1)
        ic = _inv_cnt(c_ref[...], o_ref)
        ic_ref[...] = ic
        mean = a * ic
        v = lax.dot_general(mean, wl_ref[...], (((1,), (1,)), ((), ())),
                            preferred_element_type=jnp.float32)
        h = _elu(v + b_ref[...] + r_ref[...])
        h0_ref[...] = h[:, 0 * FW:1 * FW]
        h1_ref[...] = h[:, 1 * FW:2 * FW]
        rn_ref[...] = lax.dot_general(h, wn_ref[...], (((1,), (1,)), ((), ())),
                                      preferred_element_type=jnp.float32)

    hq_spec = pl.BlockSpec((BLK, FW), lambda i: (i, 0))
    hq_shape = jax.ShapeDtypeStruct((NPAD, FW), jnp.float32)
    return pl.pallas_call(
        body,
        grid=(NPAD // BLK,),
        in_specs=[pl.BlockSpec((NQ, BLK, FW), lambda i: (0, i, 0)),
                  pl.BlockSpec((NW, BLK), lambda i: (0, i)),
                  pl.BlockSpec((NW, 1), lambda i: (0, 0)),
                  pl.BlockSpec((BLK, H), lambda i: (i, 0)),
                  pl.BlockSpec((H, H), lambda i: (0, 0)),
                  pl.BlockSpec((1, H), lambda i: (0, 0)),
                  pl.BlockSpec((H, H), lambda i: (0, 0))],
        out_specs=[hq_spec, hq_spec,
                   pl.BlockSpec((BLK, H), lambda i: (i, 0)),
                   pl.BlockSpec((BLK, 1), lambda i: (i, 0))],
        out_shape=[hq_shape, hq_shape,
                   jax.ShapeDtypeStruct((NPAD, H), jnp.float32),
                   jax.ShapeDtypeStruct((NPAD, 1), jnp.float32)],
        compiler_params=pltpu.CompilerParams(
            dimension_semantics=("parallel",)),
    )(aggp, cntp, ones32, r1, W1l, b1, W2r)


def _final_call(aggp, icnt, r2, W2l, b2):
    def body(a_ref, ic_ref, r_ref, wl_ref, b_ref, out_ref):
        av = a_ref[...]
        a = jnp.concatenate([av[0], av[1]], axis=1)
        mean = a * ic_ref[...]
        v = lax.dot_general(mean, wl_ref[...], (((1,), (1,)), ((), ())),
                            preferred_element_type=jnp.float32)
        out_ref[...] = _elu(v + b_ref[...] + r_ref[...])

    return pl.pallas_call(
        body,
        grid=(N // FBLK,),
        in_specs=[pl.BlockSpec((NQ, FBLK, FW), lambda i: (0, i, 0)),
                  pl.BlockSpec((FBLK, 1), lambda i: (i, 0)),
                  pl.BlockSpec((FBLK, H), lambda i: (i, 0)),
                  pl.BlockSpec((H, H), lambda i: (0, 0)),
                  pl.BlockSpec((1, H), lambda i: (0, 0))],
        out_specs=pl.BlockSpec((FBLK, H), lambda i: (i, 0)),
        out_shape=jax.ShapeDtypeStruct((N, H), jnp.float32),
        compiler_params=pltpu.CompilerParams(
            dimension_semantics=("parallel",)),
    )(aggp, icnt, r2, W2l, b2)


def kernel(x, knn_edge_index, W1l, b1, W1r, W2l, b2, W2r):
    src2 = knn_edge_index[0].astype(jnp.int32).reshape(ER, CW)
    dst2 = knn_edge_index[1].astype(jnp.int32).reshape(ER, CW)
    eye = jnp.eye(D, dtype=jnp.float32)
    ones32 = jnp.ones((NW, 1), jnp.float32)
    z32 = jnp.zeros((CW, FW), jnp.float32)

    t0, t1, r1, srcp, dstp = _prep_call(x, eye, W1r, src2, dst2)
    agg1, cnt1 = _sc_agg_counts(t0, t1, srcp, dstp, z32)
    cnt1p = cnt1.reshape(NW, NPAD)
    h0, h1, r2, icnt = _mid_call(agg1, cnt1p, ones32, r1, W1l,
                                 b1.reshape(1, H), W2r)
    agg2 = _sc_agg_plain(h0, h1, srcp, dstp, z32)
    return _final_call(agg2, icnt, r2, W2l, b2.reshape(1, H))
